# Initial kernel scaffold; baseline (speedup 1.0000x reference)
import functools

import jax
import jax.numpy as jnp
from jax import lax
from jax.experimental import pallas as pl
from jax.experimental.pallas import tpu as pltpu
from jax.experimental.pallas import tpu_sc as plsc

_NUM_NODES = 1000
_N = 4096
_D_SPATIAL = 125
_TOKEN_DIM = 128
_DEG = 31
_MAX_LENGTH = 64
_M = _DEG + 1

_NC = 2
_NS = 16
_NW = _NC * _NS
_TPW = _N // _NW
_L = 16
_NBUF = 4


def _tokenizer_body(data2, emb, ids, mu, sig, tim, out,
                    ids_v, idx2_v, emb_v, vals_v, tim_v, mu_v, sig_v,
                    bufs, gsem, osem):
  wid = lax.axis_index("s") * _NC + lax.axis_index("c")
  t0 = wid * _TPW

  pltpu.sync_copy(ids, ids_v)
  pltpu.sync_copy(mu, mu_v)
  pltpu.sync_copy(sig, sig_v)
  pltpu.sync_copy(tim.at[pl.ds(t0, _TPW)], tim_v)

  for g in range(_M // _L):
    iv = ids_v[pl.ds(g * _L, _L)]
    idx2_v[pl.ds(g * _L, _L)] = iv * _NW + wid

  pltpu.async_copy(emb.at[ids_v], emb_v, gsem).wait()
  pltpu.async_copy(data2.at[idx2_v], vals_v, gsem).wait()

  mu_r = mu_v[...]
  inv_s = 1.0 / sig_v[...]

  def init_emb(i, carry):
    j = i // (_TOKEN_DIM // _L)
    c = (i % (_TOKEN_DIM // _L)) * _L
    v = emb_v[j, pl.ds(c, _L)]
    for b in range(_NBUF):
      bufs[b, j, pl.ds(c, _L)] = v
    return carry
  lax.fori_loop(0, _M * (_TOKEN_DIM // _L), init_emb, 0)

  zv = jnp.zeros((_L,), jnp.float32)

  def init_zero(i, carry):
    j = _M + i // (_TOKEN_DIM // _L)
    c = (i % (_TOKEN_DIM // _L)) * _L
    for b in range(_NBUF):
      bufs[b, j, pl.ds(c, _L)] = zv
    return carry
  lax.fori_loop(0, (_MAX_LENGTH - _M) * (_TOKEN_DIM // _L), init_zero, 0)

  rows = [lax.iota(jnp.int32, _L) + g * _L for g in range(_M // _L)]
  c125 = jnp.full((_L,), 125, jnp.int32)
  c126 = jnp.full((_L,), 126, jnp.int32)
  c127 = jnp.full((_L,), 127, jnp.int32)
  zero16 = jnp.zeros((_L,), jnp.int32)
  one16 = jnp.full((_L,), 1, jnp.int32)

  def round_body(r, carry):
    for b in range(_NBUF):
      tt = r * _NBUF + b

      @pl.when(r > 0)
      def _wait():
        pltpu.make_async_copy(bufs.at[b], out.at[t0], osem.at[b]).wait()

      tts = jnp.full((_L,), tt, jnp.int32)
      t0v = plsc.load_gather(tim_v, [tts, zero16])
      t1v = plsc.load_gather(tim_v, [tts, one16])
      for g in range(_M // _L):
        v = plsc.load_gather(vals_v, [rows[g], tts])
        plsc.store_scatter(bufs.at[b], [rows[g], c125], (v - mu_r) * inv_s)
        plsc.store_scatter(bufs.at[b], [rows[g], c126], t0v)
        plsc.store_scatter(bufs.at[b], [rows[g], c127], t1v)

      pltpu.make_async_copy(bufs.at[b], out.at[t0 + tt], osem.at[b]).start()
    return carry

  lax.fori_loop(0, _TPW // _NBUF, round_body, 0)

  for b in range(_NBUF):
    pltpu.make_async_copy(bufs.at[b], out.at[t0], osem.at[b]).wait()


@jax.jit
def _tokenize(data2, emb, ids, mu, sig, tim):
  mesh = plsc.VectorSubcoreMesh(core_axis_name="c", subcore_axis_name="s",
                                num_cores=_NC, num_subcores=_NS)
  f = functools.partial(
      pl.kernel,
      out_type=jax.ShapeDtypeStruct((_N, _MAX_LENGTH, _TOKEN_DIM),
                                    jnp.float32),
      mesh=mesh,
      scratch_types=[
          pltpu.VMEM((_M,), jnp.int32),
          pltpu.VMEM((_M,), jnp.int32),
          pltpu.VMEM((_M, _TOKEN_DIM), jnp.float32),
          pltpu.VMEM((_M, _TPW), jnp.float32),
          pltpu.VMEM((_TPW, 2), jnp.float32),
          pltpu.VMEM((_L,), jnp.float32),
          pltpu.VMEM((_L,), jnp.float32),
          pltpu.VMEM((_NBUF, _MAX_LENGTH, _TOKEN_DIM), jnp.float32),
          pltpu.SemaphoreType.DMA,
          pltpu.SemaphoreType.DMA((_NBUF,)),
      ],
  )(_tokenizer_body)
  return f(data2, emb, ids, mu, sig, tim)


def kernel(data, node, spatial_embedding, temporal_all, neighbors, zt_mu,
           zt_sigma):
  data2 = data.reshape(_NUM_NODES * _NW, _TPW)
  emb = jnp.pad(spatial_embedding, ((0, 0), (0, _TOKEN_DIM - _D_SPATIAL)))
  ids = jnp.concatenate(
      [jnp.asarray(node, jnp.int32).reshape(1), neighbors.astype(jnp.int32)])
  mu = jnp.broadcast_to(zt_mu.astype(jnp.float32), (_L,))
  sig = jnp.broadcast_to(zt_sigma.astype(jnp.float32), (_L,))
  return _tokenize(data2, emb, ids, mu, sig, temporal_all)


# speedup vs baseline: 2.5921x; 2.5921x over previous
import functools

import jax
import jax.numpy as jnp
from jax import lax
from jax.experimental import pallas as pl
from jax.experimental.pallas import tpu as pltpu
from jax.experimental.pallas import tpu_sc as plsc

_NUM_NODES = 1000
_N = 4096
_D_SPATIAL = 125
_TOKEN_DIM = 128
_DEG = 31
_MAX_LENGTH = 64
_M = _DEG + 1

_NC = 2
_NS = 16
_NW = _NC * _NS
_TPW = _N // _NW
_L = 16
_NBUF = 4


def _tokenizer_body(data2, emb, ids, mu, sig, tim0, tim1, out,
                    ids_v, idx2_v, emb_v, vals_v, tim0_v, tim1_v, mu_v, sig_v,
                    bufs, gsem, osem):
  wid = lax.axis_index("s") * _NC + lax.axis_index("c")
  t0 = wid * _TPW

  pltpu.sync_copy(ids, ids_v)
  pltpu.sync_copy(mu, mu_v)
  pltpu.sync_copy(sig, sig_v)
  pltpu.sync_copy(tim0.at[pl.ds(t0, _TPW)], tim0_v)
  pltpu.sync_copy(tim1.at[pl.ds(t0, _TPW)], tim1_v)

  for g in range(_M // _L):
    iv = ids_v[pl.ds(g * _L, _L)]
    idx2_v[pl.ds(g * _L, _L)] = iv * _NW + wid

  pltpu.async_copy(emb.at[ids_v], emb_v, gsem).wait()
  pltpu.async_copy(data2.at[idx2_v], vals_v, gsem).wait()

  mu_r = mu_v[...]
  inv_s = 1.0 / sig_v[...]

  def init_emb(i, carry):
    j = i // (_TOKEN_DIM // _L)
    c = (i % (_TOKEN_DIM // _L)) * _L
    v = emb_v[j, pl.ds(c, _L)]
    for b in range(_NBUF):
      bufs[b, j, pl.ds(c, _L)] = v
    return carry
  lax.fori_loop(0, _M * (_TOKEN_DIM // _L), init_emb, 0)

  zv = jnp.zeros((_L,), jnp.float32)

  def init_zero(i, carry):
    j = _M + i // (_TOKEN_DIM // _L)
    c = (i % (_TOKEN_DIM // _L)) * _L
    for b in range(_NBUF):
      bufs[b, j, pl.ds(c, _L)] = zv
    return carry
  lax.fori_loop(0, (_MAX_LENGTH - _M) * (_TOKEN_DIM // _L), init_zero, 0)

  rows = [lax.iota(jnp.int32, _L) + g * _L for g in range(_M // _L)]
  c125 = jnp.full((_L,), 125, jnp.int32)
  c126 = jnp.full((_L,), 126, jnp.int32)
  c127 = jnp.full((_L,), 127, jnp.int32)

  def round_body(r, carry):
    for b in range(_NBUF):
      tt = r * _NBUF + b

      @pl.when(r > 0)
      def _wait():
        pltpu.make_async_copy(bufs.at[b], out.at[t0], osem.at[b]).wait()

      tts = jnp.full((_L,), tt, jnp.int32)
      t0v = plsc.load_gather(tim0_v, [tts])
      t1v = plsc.load_gather(tim1_v, [tts])
      for g in range(_M // _L):
        v = plsc.load_gather(vals_v, [rows[g], tts])
        plsc.store_scatter(bufs.at[b], [rows[g], c125], (v - mu_r) * inv_s)
        plsc.store_scatter(bufs.at[b], [rows[g], c126], t0v)
        plsc.store_scatter(bufs.at[b], [rows[g], c127], t1v)

      pltpu.make_async_copy(bufs.at[b], out.at[t0 + tt], osem.at[b]).start()
    return carry

  lax.fori_loop(0, _TPW // _NBUF, round_body, 0)

  for b in range(_NBUF):
    pltpu.make_async_copy(bufs.at[b], out.at[t0], osem.at[b]).wait()


@jax.jit
def _tokenize(data2, emb, ids, mu, sig, tim0, tim1):
  mesh = plsc.VectorSubcoreMesh(core_axis_name="c", subcore_axis_name="s",
                                num_cores=_NC, num_subcores=_NS)
  f = functools.partial(
      pl.kernel,
      out_type=jax.ShapeDtypeStruct((_N, _MAX_LENGTH, _TOKEN_DIM),
                                    jnp.float32),
      mesh=mesh,
      compiler_params=pltpu.CompilerParams(needs_layout_passes=False),
      scratch_types=[
          pltpu.VMEM((_M,), jnp.int32),
          pltpu.VMEM((_M,), jnp.int32),
          pltpu.VMEM((_M, _TOKEN_DIM), jnp.float32),
          pltpu.VMEM((_M, _TPW), jnp.float32),
          pltpu.VMEM((_TPW,), jnp.float32),
          pltpu.VMEM((_TPW,), jnp.float32),
          pltpu.VMEM((_L,), jnp.float32),
          pltpu.VMEM((_L,), jnp.float32),
          pltpu.VMEM((_NBUF, _MAX_LENGTH, _TOKEN_DIM), jnp.float32),
          pltpu.SemaphoreType.DMA,
          pltpu.SemaphoreType.DMA((_NBUF,)),
      ],
  )(_tokenizer_body)
  return f(data2, emb, ids, mu, sig, tim0, tim1)


def kernel(data, node, spatial_embedding, temporal_all, neighbors, zt_mu,
           zt_sigma):
  data2 = data.reshape(_NUM_NODES * _NW, _TPW)
  emb = jnp.pad(spatial_embedding, ((0, 0), (0, _TOKEN_DIM - _D_SPATIAL)))
  ids = jnp.concatenate(
      [jnp.asarray(node, jnp.int32).reshape(1), neighbors.astype(jnp.int32)])
  mu = jnp.broadcast_to(zt_mu.astype(jnp.float32), (_L,))
  sig = jnp.broadcast_to(zt_sigma.astype(jnp.float32), (_L,))
  tim0 = temporal_all[:, 0]
  tim1 = temporal_all[:, 1]
  return _tokenize(data2, emb, ids, mu, sig, tim0, tim1)


# speedup vs baseline: 3.2507x; 1.2541x over previous
import functools

import jax
import jax.numpy as jnp
from jax import lax
from jax.experimental import pallas as pl
from jax.experimental.pallas import tpu as pltpu
from jax.experimental.pallas import tpu_sc as plsc

_NUM_NODES = 1000
_N = 4096
_D_SPATIAL = 125
_TOKEN_DIM = 128
_DEG = 31
_MAX_LENGTH = 64
_M = _DEG + 1

_NC = 2
_NS = 16
_NW = _NC * _NS
_TPW = _N // _NW
_L = 16
_NBUF = 4


def _tokenizer_body(data, emb, ids, mu, sig, tim0, tim1, out,
                    ids_v, emb_v, vals_v, tim0_v, tim1_v, mu_v, sig_v,
                    bufs, gsem, osem):
  wid = lax.axis_index("s") * _NC + lax.axis_index("c")
  t0 = wid * _TPW

  pltpu.sync_copy(ids, ids_v)
  pltpu.sync_copy(mu, mu_v)
  pltpu.sync_copy(sig, sig_v)
  pltpu.sync_copy(tim0.at[pl.ds(t0, _TPW)], tim0_v)
  pltpu.sync_copy(tim1.at[pl.ds(t0, _TPW)], tim1_v)

  pltpu.async_copy(emb.at[ids_v], emb_v, gsem).wait()
  idv = [ids_v[pl.ds(g * _L, _L)] for g in range(_M // _L)]
  for j in range(_M):
    idj = idv[j // _L][j % _L]
    pltpu.make_async_copy(data.at[idj, pl.ds(t0, _TPW)],
                          vals_v.at[j], gsem).start()
  for j in range(_M):
    pltpu.make_async_copy(data.at[0, pl.ds(t0, _TPW)],
                          vals_v.at[j], gsem).wait()

  mu_r = mu_v[...]
  inv_s = 1.0 / sig_v[...]

  def init_emb(i, carry):
    j = i // (_TOKEN_DIM // _L)
    c = (i % (_TOKEN_DIM // _L)) * _L
    v = emb_v[j, pl.ds(c, _L)]
    for b in range(_NBUF):
      bufs[b, j, pl.ds(c, _L)] = v
    return carry
  lax.fori_loop(0, _M * (_TOKEN_DIM // _L), init_emb, 0)

  zv = jnp.zeros((_L,), jnp.float32)

  def init_zero(i, carry):
    j = _M + i // (_TOKEN_DIM // _L)
    c = (i % (_TOKEN_DIM // _L)) * _L
    for b in range(_NBUF):
      bufs[b, j, pl.ds(c, _L)] = zv
    return carry
  lax.fori_loop(0, (_MAX_LENGTH - _M) * (_TOKEN_DIM // _L), init_zero, 0)

  rows = [lax.iota(jnp.int32, _L) + g * _L for g in range(_M // _L)]
  c125 = jnp.full((_L,), 125, jnp.int32)
  c126 = jnp.full((_L,), 126, jnp.int32)
  c127 = jnp.full((_L,), 127, jnp.int32)

  def round_body(r, carry):
    for b in range(_NBUF):
      tt = r * _NBUF + b

      @pl.when(r > 0)
      def _wait():
        pltpu.make_async_copy(bufs.at[b], out.at[t0], osem.at[b]).wait()

      tts = jnp.full((_L,), tt, jnp.int32)
      t0v = plsc.load_gather(tim0_v, [tts])
      t1v = plsc.load_gather(tim1_v, [tts])
      for g in range(_M // _L):
        v = plsc.load_gather(vals_v, [rows[g], tts])
        plsc.store_scatter(bufs.at[b], [rows[g], c125], (v - mu_r) * inv_s)
        plsc.store_scatter(bufs.at[b], [rows[g], c126], t0v)
        plsc.store_scatter(bufs.at[b], [rows[g], c127], t1v)

      pltpu.make_async_copy(bufs.at[b], out.at[t0 + tt], osem.at[b]).start()
    return carry

  lax.fori_loop(0, _TPW // _NBUF, round_body, 0)

  for b in range(_NBUF):
    pltpu.make_async_copy(bufs.at[b], out.at[t0], osem.at[b]).wait()


@jax.jit
def _tokenize(data, emb, ids, mu, sig, tim0, tim1):
  mesh = plsc.VectorSubcoreMesh(core_axis_name="c", subcore_axis_name="s",
                                num_cores=_NC, num_subcores=_NS)
  f = functools.partial(
      pl.kernel,
      out_type=jax.ShapeDtypeStruct((_N, _MAX_LENGTH, _TOKEN_DIM),
                                    jnp.float32),
      mesh=mesh,
      compiler_params=pltpu.CompilerParams(needs_layout_passes=False),
      scratch_types=[
          pltpu.VMEM((_M,), jnp.int32),
          pltpu.VMEM((_M, _TOKEN_DIM), jnp.float32),
          pltpu.VMEM((_M, _TPW), jnp.float32),
          pltpu.VMEM((_TPW,), jnp.float32),
          pltpu.VMEM((_TPW,), jnp.float32),
          pltpu.VMEM((_L,), jnp.float32),
          pltpu.VMEM((_L,), jnp.float32),
          pltpu.VMEM((_NBUF, _MAX_LENGTH, _TOKEN_DIM), jnp.float32),
          pltpu.SemaphoreType.DMA,
          pltpu.SemaphoreType.DMA((_NBUF,)),
      ],
  )(_tokenizer_body)
  return f(data, emb, ids, mu, sig, tim0, tim1)


def kernel(data, node, spatial_embedding, temporal_all, neighbors, zt_mu,
           zt_sigma):
  emb = jnp.pad(spatial_embedding, ((0, 0), (0, _TOKEN_DIM - _D_SPATIAL)))
  ids = jnp.concatenate(
      [jnp.asarray(node, jnp.int32).reshape(1), neighbors.astype(jnp.int32)])
  mu = jnp.broadcast_to(zt_mu.astype(jnp.float32), (_L,))
  sig = jnp.broadcast_to(zt_sigma.astype(jnp.float32), (_L,))
  tim0 = temporal_all[:, 0]
  tim1 = temporal_all[:, 1]
  return _tokenize(data, emb, ids, mu, sig, tim0, tim1)
